# baseline (speedup 1.0000x reference)
import jax
import jax.numpy as jnp
from jax.experimental import pallas as pl
from jax.experimental.pallas import tpu as pltpu

_TN = 512


def _vq_kernel(zn_ref, z_ref, cn_ref, cb_ref, idx_ref, md_ref, q_ref):
    kK = cb_ref.shape[0]
    h = kK // 2
    z = z_ref[...]
    cb = cb_ref[...]
    mm = jax.lax.dot_general(
        z, cb, (((1,), (1,)), ((), ())),
        preferred_element_type=jnp.float32)
    dist = (zn_ref[...] + cn_ref[...]) - 2.0 * mm

    d0 = dist[:, :h]
    d1 = dist[:, h:]
    m0 = jnp.min(d0, axis=1, keepdims=True)
    m1 = jnp.min(d1, axis=1, keepdims=True)
    iota = jax.lax.broadcasted_iota(jnp.int32, d0.shape, 1)
    i0 = jnp.min(jnp.where(d0 == m0, iota, h), axis=1, keepdims=True)
    i1 = jnp.min(jnp.where(d1 == m1, iota, h), axis=1, keepdims=True) + h

    m0bf = m0.astype(jnp.bfloat16).astype(jnp.float32)
    take1 = m1 < m0bf
    idx = jnp.where(take1, i1, i0)
    md = jnp.where(take1, m1, m0)
    idx_ref[...] = idx
    md_ref[...] = md

    onehot = (jax.lax.broadcasted_iota(jnp.int32, dist.shape, 1)
              == idx).astype(jnp.float32)
    q_ref[...] = jax.lax.dot_general(
        onehot, cb, (((1,), (0,)), ((), ())),
        preferred_element_type=jnp.float32,
        precision=jax.lax.Precision.HIGHEST)


def kernel(z, codebook):
    kK, dD = codebook.shape
    z_flat = jnp.reshape(z, (-1, dD))
    n = z_flat.shape[0]
    zn = jnp.sum(z_flat ** 2, axis=1)[:, None]
    cn = jnp.sum(codebook ** 2, axis=1)[None, :]

    grid = n // _TN
    idx, md, q = pl.pallas_call(
        _vq_kernel,
        grid=(grid,),
        in_specs=[
            pl.BlockSpec((_TN, 1), lambda i: (i, 0)),
            pl.BlockSpec((_TN, dD), lambda i: (i, 0)),
            pl.BlockSpec((1, kK), lambda i: (0, 0)),
            pl.BlockSpec((kK, dD), lambda i: (0, 0)),
        ],
        out_specs=[
            pl.BlockSpec((_TN, 1), lambda i: (i, 0)),
            pl.BlockSpec((_TN, 1), lambda i: (i, 0)),
            pl.BlockSpec((_TN, dD), lambda i: (i, 0)),
        ],
        out_shape=[
            jax.ShapeDtypeStruct((n, 1), jnp.int32),
            jax.ShapeDtypeStruct((n, 1), jnp.float32),
            jax.ShapeDtypeStruct((n, dD), jnp.float32),
        ],
    )(zn, z_flat, cn, codebook)

    quantized = q.reshape(z.shape)
    loss = 1.25 * (jnp.sum(md) / jnp.float32(z.size))
    quantized_st = z + (quantized - z)
    return (quantized_st, loss)


# speedup vs baseline: 1.5195x; 1.5195x over previous
import jax
import jax.numpy as jnp
from jax.experimental import pallas as pl
from jax.experimental.pallas import tpu as pltpu

_TN = 512


def _vq_kernel(zn_ref, z_ref, cn_ref, cb_ref, cbh_ref, cbl_ref,
               idx_ref, md_ref, q_ref):
    kK = cb_ref.shape[0]
    h = kK // 2
    z = z_ref[...]
    cb = cb_ref[...]
    mm2 = jax.lax.dot_general(
        -2.0 * z, cb, (((1,), (1,)), ((), ())),
        preferred_element_type=jnp.float32)
    dist = (zn_ref[...] + cn_ref[...]) + mm2

    d0 = dist[:, :h]
    d1 = dist[:, h:]
    m0 = jnp.min(d0, axis=1, keepdims=True)
    m1 = jnp.min(d1, axis=1, keepdims=True)
    iota = jax.lax.broadcasted_iota(jnp.int32, d0.shape, 1)
    i0 = jnp.min(jnp.where(d0 == m0, iota, h), axis=1, keepdims=True)
    i1 = jnp.min(jnp.where(d1 == m1, iota, h), axis=1, keepdims=True) + h

    m0bf = m0.astype(jnp.bfloat16).astype(jnp.float32)
    take1 = m1 < m0bf
    idx = jnp.where(take1, i1, i0)
    md = jnp.where(take1, m1, m0)
    idx_ref[...] = idx
    md_ref[...] = md

    onehot = (jax.lax.broadcasted_iota(jnp.int32, dist.shape, 1)
              == idx).astype(jnp.float32)
    dn = (((1,), (0,)), ((), ()))
    q_ref[...] = (
        jax.lax.dot_general(onehot, cbh_ref[...], dn,
                            preferred_element_type=jnp.float32)
        + jax.lax.dot_general(onehot, cbl_ref[...], dn,
                              preferred_element_type=jnp.float32))


def kernel(z, codebook):
    kK, dD = codebook.shape
    z_flat = jnp.reshape(z, (-1, dD))
    n = z_flat.shape[0]
    zn = jnp.sum(z_flat ** 2, axis=1)[:, None]
    cn = jnp.sum(codebook ** 2, axis=1)[None, :]
    cb_hi = codebook.astype(jnp.bfloat16).astype(jnp.float32)
    cb_lo = codebook - cb_hi

    grid = n // _TN
    idx, md, q = pl.pallas_call(
        _vq_kernel,
        grid=(grid,),
        in_specs=[
            pl.BlockSpec((_TN, 1), lambda i: (i, 0)),
            pl.BlockSpec((_TN, dD), lambda i: (i, 0)),
            pl.BlockSpec((1, kK), lambda i: (0, 0)),
            pl.BlockSpec((kK, dD), lambda i: (0, 0)),
            pl.BlockSpec((kK, dD), lambda i: (0, 0)),
            pl.BlockSpec((kK, dD), lambda i: (0, 0)),
        ],
        out_specs=[
            pl.BlockSpec((_TN, 1), lambda i: (i, 0)),
            pl.BlockSpec((_TN, 1), lambda i: (i, 0)),
            pl.BlockSpec((_TN, dD), lambda i: (i, 0)),
        ],
        out_shape=[
            jax.ShapeDtypeStruct((n, 1), jnp.int32),
            jax.ShapeDtypeStruct((n, 1), jnp.float32),
            jax.ShapeDtypeStruct((n, dD), jnp.float32),
        ],
    )(zn, z_flat, cn, codebook, cb_hi, cb_lo)

    quantized = q.reshape(z.shape)
    loss = 1.25 * (jnp.sum(md) / jnp.float32(z.size))
    quantized_st = z + (quantized - z)
    return (quantized_st, loss)


# speedup vs baseline: 2.8979x; 1.9071x over previous
import functools

import jax
import jax.numpy as jnp
from jax.experimental import pallas as pl
from jax.experimental.pallas import tpu as pltpu
from jax.experimental.pallas import tpu_sc as plsc

_TN = 512


def _vq_kernel(zn_ref, z_ref, cn_ref, cb_ref, idx_ref, ls_ref):
    kK = cb_ref.shape[0]
    h = kK // 2
    z = z_ref[...]
    cb = cb_ref[...]
    mm2 = jax.lax.dot_general(
        -2.0 * z, cb, (((1,), (1,)), ((), ())),
        preferred_element_type=jnp.float32)
    dist = (zn_ref[...] + cn_ref[...]) + mm2

    d0 = dist[:, :h]
    d1 = dist[:, h:]
    m0 = jnp.min(d0, axis=1, keepdims=True)
    m1 = jnp.min(d1, axis=1, keepdims=True)
    iota = jax.lax.broadcasted_iota(jnp.int32, d0.shape, 1)
    i0 = jnp.min(jnp.where(d0 == m0, iota, h), axis=1, keepdims=True)
    i1 = jnp.min(jnp.where(d1 == m1, iota, h), axis=1, keepdims=True) + h

    m0bf = m0.astype(jnp.bfloat16).astype(jnp.float32)
    take1 = m1 < m0bf
    idx_ref[...] = jnp.where(take1, i1, i0)
    md = jnp.where(take1, m1, m0)

    @pl.when(pl.program_id(0) == 0)
    def _init():
        ls_ref[...] = jnp.zeros_like(ls_ref)

    ls_ref[...] += jnp.sum(md).reshape(1, 1)


def _tc_argmin(z_flat, codebook, zn, cn):
    kK, dD = codebook.shape
    n = z_flat.shape[0]
    return pl.pallas_call(
        _vq_kernel,
        grid=(n // _TN,),
        in_specs=[
            pl.BlockSpec((_TN, 1), lambda i: (i, 0)),
            pl.BlockSpec((_TN, dD), lambda i: (i, 0)),
            pl.BlockSpec((1, kK), lambda i: (0, 0)),
            pl.BlockSpec((kK, dD), lambda i: (0, 0)),
        ],
        out_specs=[
            pl.BlockSpec((_TN, 1), lambda i: (i, 0)),
            pl.BlockSpec((1, 1), lambda i: (0, 0)),
        ],
        out_shape=[
            jax.ShapeDtypeStruct((n, 1), jnp.int32),
            jax.ShapeDtypeStruct((1, 1), jnp.float32),
        ],
    )(zn, z_flat, cn, codebook)


def _sc_gather(codebook, idx_flat):
    info = plsc.get_sparse_core_info()
    nw = info.num_cores * info.num_subcores
    n = idx_flat.shape[0]
    dD = codebook.shape[1]
    if dD % 128:
        codebook = jnp.pad(codebook, ((0, 0), (0, 128 - dD)))
    dP = codebook.shape[1]
    b_per_w = n // nw
    mesh = plsc.VectorSubcoreMesh(core_axis_name="c", subcore_axis_name="s")

    @functools.partial(
        pl.kernel, mesh=mesh,
        out_type=jax.ShapeDtypeStruct((n, dP), jnp.float32),
        scratch_types=[
            pltpu.VMEM((b_per_w,), jnp.int32),
            pltpu.VMEM((b_per_w, dP), jnp.float32),
            pltpu.SemaphoreType.DMA,
        ],
    )
    def k(table_hbm, idx_hbm, out_hbm, idx_v, rows_v, sem):
        wid = jax.lax.axis_index("s") * info.num_cores + jax.lax.axis_index("c")
        base = wid * b_per_w
        pltpu.sync_copy(idx_hbm.at[pl.ds(base, b_per_w)], idx_v)
        pltpu.async_copy(table_hbm.at[idx_v], rows_v, sem).wait()
        pltpu.sync_copy(rows_v, out_hbm.at[pl.ds(base, b_per_w)])

    return k(codebook, idx_flat)[:, :dD]


def kernel(z, codebook):
    kK, dD = codebook.shape
    z_flat = jnp.reshape(z, (-1, dD))
    zn = jnp.sum(z_flat ** 2, axis=1)[:, None]
    cn = jnp.sum(codebook ** 2, axis=1)[None, :]

    idx, lsum = _tc_argmin(z_flat, codebook, zn, cn)
    quantized = _sc_gather(codebook, idx.reshape(-1)).reshape(z.shape)

    loss = 1.25 * (lsum[0, 0] / jnp.float32(z.size))
    quantized_st = z + (quantized - z)
    return (quantized_st, loss)
